# speedup vs baseline: 2.4699x; 2.4699x over previous
import jax
import jax.numpy as jnp
from jax.experimental import pallas as pl

N = 512
B = 64
L = 12
U = 64
K = 2
NUM_MAT = K + 1


def _layer_body(xin_ref, h_ref, s_ref, wg_ref, bg_ref, wc_ref, bc_ref, out_ref):
    s = s_ref[...]
    xin = xin_ref[0]
    h = h_ref[0]

    wg = wg_ref[...]
    bg = bg_ref[...]
    wc = wc_ref[...]
    bc = bc_ref[...]

    g0 = jnp.concatenate([xin, h], axis=1)
    g1 = jnp.dot(s, g0, preferred_element_type=jnp.float32)
    g2 = 2.0 * jnp.dot(s, g1, preferred_element_type=jnp.float32) - g0
    val = (jnp.dot(g0, wg[0], preferred_element_type=jnp.float32)
           + jnp.dot(g1, wg[1], preferred_element_type=jnp.float32)
           + jnp.dot(g2, wg[2], preferred_element_type=jnp.float32)
           + bg)
    val = jax.nn.sigmoid(val)
    r = val[:, :U]
    u = val[:, U:]

    c0 = jnp.concatenate([xin, r * h], axis=1)
    c1 = jnp.dot(s, c0, preferred_element_type=jnp.float32)
    c2 = 2.0 * jnp.dot(s, c1, preferred_element_type=jnp.float32) - c0
    c = jnp.tanh(jnp.dot(c0, wc[0], preferred_element_type=jnp.float32)
                 + jnp.dot(c1, wc[1], preferred_element_type=jnp.float32)
                 + jnp.dot(c2, wc[2], preferred_element_type=jnp.float32)
                 + bc)

    out_ref[0] = u * h + (1.0 - u) * c


def _dcgru_layer(x_in, h, support, Wg, bg, Wc, bc):
    F = x_in.shape[-1]
    in_sz = F + U
    Wg3 = Wg.reshape(in_sz, NUM_MAT, 2 * U).transpose(1, 0, 2)
    Wc3 = Wc.reshape(in_sz, NUM_MAT, U).transpose(1, 0, 2)
    bg2 = bg.reshape(1, 2 * U)
    bc2 = bc.reshape(1, U)

    return pl.pallas_call(
        _layer_body,
        grid=(B,),
        in_specs=[
            pl.BlockSpec((1, N, F), lambda b: (b, 0, 0)),
            pl.BlockSpec((1, N, U), lambda b: (b, 0, 0)),
            pl.BlockSpec((N, N), lambda b: (0, 0)),
            pl.BlockSpec((NUM_MAT, in_sz, 2 * U), lambda b: (0, 0, 0)),
            pl.BlockSpec((1, 2 * U), lambda b: (0, 0)),
            pl.BlockSpec((NUM_MAT, in_sz, U), lambda b: (0, 0, 0)),
            pl.BlockSpec((1, U), lambda b: (0, 0)),
        ],
        out_specs=pl.BlockSpec((1, N, U), lambda b: (b, 0, 0)),
        out_shape=jax.ShapeDtypeStruct((B, N, U), jnp.float32),
    )(x_in, h, support, Wg3, bg2, Wc3, bc2)


@jax.jit
def kernel(inputs, hidden_state, support, Wg0, bg0, Wc0, bc0, Wg1, bg1, Wc1, bc1):
    x = inputs.reshape(B, N, L)
    h0_in = hidden_state[0].reshape(B, N, U)
    h1_in = hidden_state[1].reshape(B, N, U)
    h0 = _dcgru_layer(x, h0_in, support, Wg0, bg0, Wc0, bc0)
    h1 = _dcgru_layer(h0, h1_in, support, Wg1, bg1, Wc1, bc1)
    h0f = h0.reshape(B, N * U)
    h1f = h1.reshape(B, N * U)
    return h1f, jnp.stack([h0f, h1f], axis=0)
